# baseline (speedup 1.0000x reference)
import functools

import jax
import jax.numpy as jnp
from jax import lax
from jax.experimental import pallas as pl
from jax.experimental.pallas import tpu as pltpu
from jax.experimental.pallas import tpu_sc as plsc

NC = 2
NS = 16
NW = NC * NS
LANES = 16


def _emb_body(s_per_w, rows_chunk, n_chunks, d,
              ids_hbm, tok_hbm, pos_hbm, out_hbm,
              idx_v, tok_v, pos_v,
              gsem0, gsem1, psem0, psem1, ssem0, ssem1):
    b_per_w = rows_chunk * n_chunks
    wid = lax.axis_index("s") * NC + lax.axis_index("c")
    base = wid * b_per_w
    pos_base = (wid % s_per_w) * b_per_w

    gsems = (gsem0, gsem1)
    psems = (psem0, psem1)
    ssems = (ssem0, ssem1)

    pltpu.sync_copy(ids_hbm.at[pl.ds(base, b_per_w)], idx_v)

    def start_fetch(k):
        b = k % 2
        g = pltpu.async_copy(
            tok_hbm.at[idx_v.at[pl.ds(k * rows_chunk, rows_chunk)]],
            tok_v.at[b], gsems[b])
        p = pltpu.async_copy(
            pos_hbm.at[pl.ds(pos_base + k * rows_chunk, rows_chunk)],
            pos_v.at[b], psems[b])
        return g, p

    fetches = {0: start_fetch(0)}
    stores = {}
    for k in range(n_chunks):
        b = k % 2
        if k >= 1:
            stores.pop(k - 1).wait()
        if k + 1 < n_chunks:
            fetches[k + 1] = start_fetch(k + 1)
        g, p = fetches.pop(k)
        g.wait()
        p.wait()

        def row_body(r, carry):
            for c in range(d // LANES):
                sl = pl.ds(c * LANES, LANES)
                plsc.addupdate(tok_v.at[b, r, sl], pos_v[b, r, sl])
            return carry

        lax.fori_loop(0, rows_chunk, row_body, 0)
        stores[k] = pltpu.async_copy(
            tok_v.at[b],
            out_hbm.at[pl.ds(base + k * rows_chunk, rows_chunk)], ssems[b])
    stores.pop(n_chunks - 1).wait()


def kernel(input_ids, token_table, pos_table):
    batch, seq = input_ids.shape
    vocab, d = token_table.shape
    n = batch * seq
    ids_flat = input_ids.reshape(n).astype(jnp.int32)

    b_per_w = n // NW
    n_chunks = 8
    rows_chunk = b_per_w // n_chunks
    s_per_w = seq // b_per_w

    mesh = plsc.VectorSubcoreMesh(core_axis_name="c", subcore_axis_name="s")

    run = functools.partial(
        pl.kernel,
        mesh=mesh,
        out_type=jax.ShapeDtypeStruct((n, d), jnp.float32),
        scratch_types=[
            pltpu.VMEM((b_per_w,), jnp.int32),
            pltpu.VMEM((2, rows_chunk, d), jnp.float32),
            pltpu.VMEM((2, rows_chunk, d), jnp.float32),
            pltpu.SemaphoreType.DMA,
            pltpu.SemaphoreType.DMA,
            pltpu.SemaphoreType.DMA,
            pltpu.SemaphoreType.DMA,
            pltpu.SemaphoreType.DMA,
            pltpu.SemaphoreType.DMA,
        ],
    )(functools.partial(_emb_body, s_per_w, rows_chunk, n_chunks, d))

    out = run(ids_flat, token_table, pos_table)
    return out.reshape(batch, seq, d)


# speedup vs baseline: 1.0515x; 1.0515x over previous
import functools

import jax
import jax.numpy as jnp
from jax import lax
from jax.experimental import pallas as pl
from jax.experimental.pallas import tpu as pltpu
from jax.experimental.pallas import tpu_sc as plsc

NC = 2
NS = 16
NW = NC * NS
LANES = 16


def _emb_body(batch, seq, s_per_w, rows_chunk, d,
              ids_hbm, tok_hbm, pos_hbm, out_hbm,
              idx_v, tok_v, pos_v,
              gsem0, gsem1, ssem0, ssem1, psem):
    cpb = s_per_w // rows_chunk
    n_chunks = batch * cpb
    wid = lax.axis_index("s") * NC + lax.axis_index("c")
    seq0 = wid * s_per_w

    gsems = (gsem0, gsem1)
    ssems = (ssem0, ssem1)

    pos_fetch = pltpu.async_copy(pos_hbm.at[pl.ds(seq0, s_per_w)], pos_v, psem)
    for b in range(batch):
        pltpu.sync_copy(
            ids_hbm.at[pl.ds(b * seq + seq0, s_per_w)], idx_v.at[b])

    def chunk_coords(k):
        b, h = divmod(k, cpb)
        row0 = b * seq + seq0 + h * rows_chunk
        return b, h, row0

    def start_gather(k):
        b, h, _ = chunk_coords(k)
        buf = k % 2
        return pltpu.async_copy(
            tok_hbm.at[idx_v.at[b, pl.ds(h * rows_chunk, rows_chunk)]],
            tok_v.at[buf], gsems[buf])

    fetches = {0: start_gather(0)}
    stores = {}
    for k in range(n_chunks):
        buf = k % 2
        _, h, row0 = chunk_coords(k)
        if k >= 1:
            stores.pop(k - 1).wait()
        if k + 1 < n_chunks:
            fetches[k + 1] = start_gather(k + 1)
        fetches.pop(k).wait()
        if k == 0:
            pos_fetch.wait()

        def row_body(r, carry):
            for c in range(d // LANES):
                sl = pl.ds(c * LANES, LANES)
                plsc.addupdate(tok_v.at[buf, r, sl],
                               pos_v[h * rows_chunk + r, sl])
            return carry

        lax.fori_loop(0, rows_chunk, row_body, 0)
        stores[k] = pltpu.async_copy(
            tok_v.at[buf], out_hbm.at[pl.ds(row0, rows_chunk)], ssems[buf])
    stores.pop(n_chunks - 1).wait()


def kernel(input_ids, token_table, pos_table):
    batch, seq = input_ids.shape
    vocab, d = token_table.shape
    n = batch * seq
    ids_flat = input_ids.reshape(n).astype(jnp.int32)

    s_per_w = seq // NW
    rows_chunk = 32

    mesh = plsc.VectorSubcoreMesh(core_axis_name="c", subcore_axis_name="s")

    run = functools.partial(
        pl.kernel,
        mesh=mesh,
        out_type=jax.ShapeDtypeStruct((n, d), jnp.float32),
        scratch_types=[
            pltpu.VMEM((batch, s_per_w), jnp.int32),
            pltpu.VMEM((2, rows_chunk, d), jnp.float32),
            pltpu.VMEM((s_per_w, d), jnp.float32),
            pltpu.SemaphoreType.DMA,
            pltpu.SemaphoreType.DMA,
            pltpu.SemaphoreType.DMA,
            pltpu.SemaphoreType.DMA,
            pltpu.SemaphoreType.DMA,
        ],
    )(functools.partial(_emb_body, batch, seq, s_per_w, rows_chunk, d))

    out = run(ids_flat, token_table, pos_table)
    return out.reshape(batch, seq, d)


# speedup vs baseline: 1.2398x; 1.1791x over previous
import functools

import jax
import jax.numpy as jnp
from jax import lax
from jax.experimental import pallas as pl
from jax.experimental.pallas import tpu as pltpu
from jax.experimental.pallas import tpu_sc as plsc

NC = 2
NS = 16
NW = NC * NS
LANES = 16


def _emb_body(batch, seq, s_per_w, rows_chunk, d,
              ids_hbm, tok_hbm, pos_hbm, out_hbm,
              idx_v, tok_v, pos_v,
              gsem0, gsem1, ssem0, ssem1, psem):
    cpb = s_per_w // rows_chunk
    n_chunks = batch * cpb
    wid = lax.axis_index("s") * NC + lax.axis_index("c")
    seq0 = wid * s_per_w

    gsems = (gsem0, gsem1)
    ssems = (ssem0, ssem1)

    pos_fetch = pltpu.async_copy(pos_hbm.at[pl.ds(seq0, s_per_w)], pos_v, psem)
    for b in range(batch):
        pltpu.sync_copy(
            ids_hbm.at[pl.ds(b * seq + seq0, s_per_w)], idx_v.at[b])

    def chunk_coords(k):
        b, h = divmod(k, cpb)
        row0 = b * seq + seq0 + h * rows_chunk
        return b, h, row0

    def start_gather(k):
        b, h, _ = chunk_coords(k)
        buf = k % 2
        return pltpu.async_copy(
            tok_hbm.at[idx_v.at[b, pl.ds(h * rows_chunk, rows_chunk)]],
            tok_v.at[buf], gsems[buf])

    fetches = {0: start_gather(0)}
    stores = {}
    for k in range(n_chunks):
        buf = k % 2
        _, h, row0 = chunk_coords(k)
        if k >= 1:
            stores.pop(k - 1).wait()
        if k + 1 < n_chunks:
            fetches[k + 1] = start_gather(k + 1)
        fetches.pop(k).wait()
        if k == 0:
            pos_fetch.wait()

        @plsc.parallel_loop(0, rows_chunk, unroll=2)
        def row_body(r):
            for c in range(d // LANES):
                sl = pl.ds(c * LANES, LANES)
                plsc.addupdate(tok_v.at[buf, r, sl],
                               pos_v[h * rows_chunk + r, sl])
        stores[k] = pltpu.async_copy(
            tok_v.at[buf], out_hbm.at[pl.ds(row0, rows_chunk)], ssems[buf])
    stores.pop(n_chunks - 1).wait()


def kernel(input_ids, token_table, pos_table):
    batch, seq = input_ids.shape
    vocab, d = token_table.shape
    n = batch * seq
    ids_flat = input_ids.reshape(n).astype(jnp.int32)

    s_per_w = seq // NW
    rows_chunk = 32

    mesh = plsc.VectorSubcoreMesh(core_axis_name="c", subcore_axis_name="s")

    run = functools.partial(
        pl.kernel,
        mesh=mesh,
        out_type=jax.ShapeDtypeStruct((n, d), jnp.float32),
        scratch_types=[
            pltpu.VMEM((batch, s_per_w), jnp.int32),
            pltpu.VMEM((2, rows_chunk, d), jnp.float32),
            pltpu.VMEM((s_per_w, d), jnp.float32),
            pltpu.SemaphoreType.DMA,
            pltpu.SemaphoreType.DMA,
            pltpu.SemaphoreType.DMA,
            pltpu.SemaphoreType.DMA,
            pltpu.SemaphoreType.DMA,
        ],
    )(functools.partial(_emb_body, batch, seq, s_per_w, rows_chunk, d))

    out = run(ids_flat, token_table, pos_table)
    return out.reshape(batch, seq, d)
